# baseline (speedup 1.0000x reference)
import functools

import jax
import jax.numpy as jnp
from jax.experimental import pallas as pl
from jax.experimental.pallas import tpu as pltpu


def _moe_block_kernel(nexp, xc_ref, wyr_ref, wyi_ref, s_ref,
                      cnt_ref, bias_ref, actr_ref, acti_ref):
    e = pl.program_id(1)
    xc = xc_ref[...]
    yr = jnp.dot(xc, wyr_ref[0], preferred_element_type=jnp.float32)
    yi = jnp.dot(xc, wyi_ref[0], preferred_element_type=jnp.float32)
    onehot = (jax.lax.broadcasted_iota(jnp.int32, (nexp, 1), 0) == e
              ).astype(jnp.float32)
    s = jnp.dot(s_ref[...], onehot,
                preferred_element_type=jnp.float32)

    @pl.when(e == 0)
    def _init():
        actr_ref[...] = s * yr
        acti_ref[...] = s * yi

    @pl.when(e != 0)
    def _acc():
        actr_ref[...] += s * yr
        acti_ref[...] += s * yi

    @pl.when(e == nexp - 1)
    def _finalize():
        cnt = jnp.maximum(cnt_ref[...], 1.0)
        outr = actr_ref[...] / cnt
        outi = acti_ref[...] / cnt
        mag = jnp.sqrt(outr * outr + outi * outi)
        safe = jnp.maximum(mag, 1e-8)
        scale = jax.nn.relu(mag + bias_ref[...]) / safe
        actr_ref[...] = outr * scale
        acti_ref[...] = outi * scale


def kernel(x, gate_weights, experts_weight, modrelu_bias):
    B, D, _ = x.shape
    E = gate_weights.shape[1]
    k = max(1, B // E)

    xg = x.reshape(B, 2 * D)
    scores = jnp.matmul(xg, gate_weights)
    st, ti = jax.lax.top_k(scores.T, k)
    topk_scores = st.T
    topk_indices = ti.T

    eidx = jnp.arange(E)[:, None]
    sel = jnp.zeros((E, B), jnp.float32).at[eidx, ti].set(st)
    cnt = jnp.zeros((B,), jnp.float32).at[ti.reshape(-1)].add(1.0)
    s_dense = sel.T
    cnt2 = cnt[:, None]

    xr = x[..., 0].astype(jnp.bfloat16)
    xi = x[..., 1].astype(jnp.bfloat16)
    xc = jnp.concatenate([xr, xi], axis=1)
    wr = experts_weight[..., 0].astype(jnp.bfloat16)
    wi = experts_weight[..., 1].astype(jnp.bfloat16)
    wyr = jnp.concatenate([wr, -wi], axis=1)
    wyi = jnp.concatenate([wi, wr], axis=1)
    bias2 = modrelu_bias[None, :]

    BLK = min(2048, B)
    nb = B // BLK
    grid = (nb, E)
    out_shapes = (
        jax.ShapeDtypeStruct((B, D), jnp.float32),
        jax.ShapeDtypeStruct((B, D), jnp.float32),
    )
    actr, acti = pl.pallas_call(
        functools.partial(_moe_block_kernel, E),
        grid=grid,
        in_specs=[
            pl.BlockSpec((BLK, 2 * D), lambda i, e: (i, 0)),
            pl.BlockSpec((1, 2 * D, D), lambda i, e: (e, 0, 0)),
            pl.BlockSpec((1, 2 * D, D), lambda i, e: (e, 0, 0)),
            pl.BlockSpec((BLK, E), lambda i, e: (i, 0)),
            pl.BlockSpec((BLK, 1), lambda i, e: (i, 0)),
            pl.BlockSpec((1, D), lambda i, e: (0, 0)),
        ],
        out_specs=(
            pl.BlockSpec((BLK, D), lambda i, e: (i, 0)),
            pl.BlockSpec((BLK, D), lambda i, e: (i, 0)),
        ),
        out_shape=out_shapes,
        compiler_params=pltpu.CompilerParams(
            dimension_semantics=("parallel", "arbitrary"),
        ),
    )(xc, wyr, wyi, s_dense, cnt2, bias2)

    act = jnp.stack([actr, acti], axis=-1)
    counts = cnt2.reshape(B, 1, 1)
    return (act, topk_indices, topk_scores, counts)
